# baseline (speedup 1.0000x reference)
import functools

import jax
import jax.numpy as jnp
from jax import lax
from jax.experimental import pallas as pl
from jax.experimental.pallas import tpu as pltpu
from jax.experimental.pallas import tpu_sc as plsc

N = 10000
E = 320000
D = 128
NCLS = 40

NC = 2
NS = 16
NW = NC * NS
CHUNK = 40
RPW = 256
EPAD = NW * RPW * CHUNK
NPAD = 10112
RPT = NPAD // NS
NSLOT = 8
GRP = 8
NGRP = RPW // GRP
FILLB = 48


def _chunks(total, maxsz):
    out = [(i * maxsz, maxsz) for i in range(total // maxsz)]
    if total % maxsz:
        out.append(((total // maxsz) * maxsz, total % maxsz))
    return out


_ROW_CHUNKS = _chunks(RPT, CHUNK)

_f32 = jnp.float32


def _sc_degrees(src3d, dst3d):
    mesh = plsc.VectorSubcoreMesh(core_axis_name="c", subcore_axis_name="s")

    @functools.partial(
        pl.kernel,
        mesh=mesh,
        out_type=(
            jax.ShapeDtypeStruct((NC * NPAD,), _f32),
            jax.ShapeDtypeStruct((NC * NPAD,), _f32),
        ),
        scratch_types=[
            pltpu.VMEM((RPW, CHUNK), jnp.int32),
            pltpu.VMEM((RPW, CHUNK), jnp.int32),
            pltpu.VMEM((FILLB,), _f32),
            pltpu.VMEM((FILLB,), _f32),
            pltpu.VMEM_SHARED((NPAD,), _f32),
            pltpu.VMEM_SHARED((NPAD,), _f32),
            pltpu.SemaphoreType.DMA,
            pltpu.SemaphoreType.DMA,
        ],
    )
    def deg_kernel(src_hbm, dst_hbm, outs_hbm, outd_hbm,
                   sidx, didx, ones, zeros, acc_s, acc_d, sem_s, sem_d):
        c = lax.axis_index("c")
        s = lax.axis_index("s")
        w = c * NS + s

        pltpu.sync_copy(src_hbm.at[w], sidx)
        pltpu.sync_copy(dst_hbm.at[w], didx)
        one16 = jnp.full((16,), 1.0, _f32)
        zero16 = jnp.zeros((16,), _f32)

        def fill(i, _):
            ones[pl.ds(i * 16, 16)] = one16
            zeros[pl.ds(i * 16, 16)] = zero16
            return 0

        lax.fori_loop(0, FILLB // 16, fill, 0)

        r0 = s * RPT
        for off, size in _ROW_CHUNKS:
            pltpu.sync_copy(zeros.at[pl.ds(0, size)],
                            acc_s.at[pl.ds(r0 + off, size)])
            pltpu.sync_copy(zeros.at[pl.ds(0, size)],
                            acc_d.at[pl.ds(r0 + off, size)])
        plsc.subcore_barrier()

        GB = 8

        def group(g, _):
            for b in range(GB):
                r = g * GB + b
                pltpu.async_copy(ones.at[pl.ds(0, CHUNK)],
                                 acc_s.at[sidx.at[r]], sem_s, add=True)
                pltpu.async_copy(ones.at[pl.ds(0, CHUNK)],
                                 acc_d.at[didx.at[r]], sem_d, add=True)
            for b in range(GB):
                r = g * GB + b
                pltpu.make_async_copy(ones.at[pl.ds(0, CHUNK)],
                                      acc_s.at[sidx.at[r]], sem_s).wait()
                pltpu.make_async_copy(ones.at[pl.ds(0, CHUNK)],
                                      acc_d.at[didx.at[r]], sem_d).wait()
            return 0

        lax.fori_loop(0, RPW // GB, group, 0)
        plsc.subcore_barrier()

        for off, size in _ROW_CHUNKS:
            row = r0 + off
            pltpu.sync_copy(acc_s.at[pl.ds(row, size)], zeros.at[pl.ds(0, size)])
            pltpu.sync_copy(zeros.at[pl.ds(0, size)],
                            outs_hbm.at[pl.ds(c * NPAD + row, size)])
            pltpu.sync_copy(acc_d.at[pl.ds(row, size)], ones.at[pl.ds(0, size)])
            pltpu.sync_copy(ones.at[pl.ds(0, size)],
                            outd_hbm.at[pl.ds(c * NPAD + row, size)])

    return deg_kernel(src3d, dst3d)


def _sc_aggregate(h, src3d, dst3d):
    mesh = plsc.VectorSubcoreMesh(core_axis_name="c", subcore_axis_name="s")

    @functools.partial(
        pl.kernel,
        mesh=mesh,
        out_type=jax.ShapeDtypeStruct((NC, NPAD, D), _f32),
        scratch_types=(
            [pltpu.VMEM((GRP, CHUNK), jnp.int32) for _ in range(4)]
            + [pltpu.VMEM((CHUNK, D), _f32) for _ in range(NSLOT)]
            + [pltpu.VMEM_SHARED((NPAD, D), _f32)]
            + [pltpu.SemaphoreType.DMA for _ in range(2 * NSLOT + 4)]
        ),
    )
    def agg_kernel(h_hbm, src_hbm, dst_hbm, out_hbm, *scr):
        sbuf = list(scr[0:2])
        dbuf = list(scr[2:4])
        rows = list(scr[4:4 + NSLOT])
        acc = scr[4 + NSLOT]
        gsem = list(scr[5 + NSLOT:5 + 2 * NSLOT])
        ssem = list(scr[5 + 2 * NSLOT:5 + 3 * NSLOT])
        isem_s = list(scr[5 + 3 * NSLOT:7 + 3 * NSLOT])
        isem_d = list(scr[7 + 3 * NSLOT:9 + 3 * NSLOT])
        c = lax.axis_index("c")
        s = lax.axis_index("s")
        w = c * NS + s

        def fire_sidx(grp, par, sync=False):
            srcslice = src_hbm.at[w, pl.ds(grp * GRP, GRP)]
            if sync:
                pltpu.sync_copy(srcslice, sbuf[par])
            else:
                pltpu.async_copy(srcslice, sbuf[par], isem_s[par])

        def fire_didx(grp, par, sync=False):
            dstslice = dst_hbm.at[w, pl.ds(grp * GRP, GRP)]
            if sync:
                pltpu.sync_copy(dstslice, dbuf[par])
            else:
                pltpu.async_copy(dstslice, dbuf[par], isem_d[par])

        def wait_sidx(par):
            pltpu.make_async_copy(src_hbm.at[w, pl.ds(0, GRP)],
                                  sbuf[par], isem_s[par]).wait()

        def wait_didx(par):
            pltpu.make_async_copy(dst_hbm.at[w, pl.ds(0, GRP)],
                                  dbuf[par], isem_d[par]).wait()

        def fire_gather(sidx_row, b):
            pltpu.async_copy(h_hbm.at[sidx_row], rows[b], gsem[b])

        def wait_gather(b):
            pltpu.make_async_copy(h_hbm.at[pl.ds(0, CHUNK)],
                                  rows[b], gsem[b]).wait()

        def fire_scatter(didx_row, b):
            pltpu.async_copy(rows[b], acc.at[didx_row], ssem[b], add=True)

        def drain_scatter(b):
            pltpu.make_async_copy(h_hbm.at[pl.ds(0, CHUNK)],
                                  rows[b], ssem[b]).wait()

        zero16 = jnp.zeros((16,), _f32)

        def zr(i, _):
            for j in range(D // 16):
                rows[0][i, pl.ds(j * 16, 16)] = zero16
            return 0

        lax.fori_loop(0, CHUNK, zr, 0)
        r0 = s * RPT
        for off, size in _ROW_CHUNKS:
            pltpu.sync_copy(rows[0].at[pl.ds(0, size)],
                            acc.at[pl.ds(r0 + off, size)])
        plsc.subcore_barrier()

        fire_sidx(0, 0, sync=True)
        fire_didx(0, 0, sync=True)
        fire_sidx(1, 1)
        for b in range(4):
            fire_gather(sbuf[0].at[b], b)

        def run_group(g, par, first_pair):
            nxt = 1 - par
            if first_pair:
                @pl.when(g > 0)
                def _():
                    wait_didx(par)
            else:
                wait_didx(par)
            for j in range(GRP):
                if j == 4:
                    @pl.when(g < NGRP - 1)
                    def _():
                        wait_sidx(nxt)
                        fire_didx(g + 1, nxt)
                b = j
                wait_gather(b)
                fire_scatter(dbuf[par].at[j], b)
                if j < 4:
                    if first_pair:
                        @pl.when(g > 0)
                        def _():
                            drain_scatter(j + 4)
                    else:
                        drain_scatter(j + 4)
                    fire_gather(sbuf[par].at[j + 4], j + 4)
                else:
                    drain_scatter(j - 4)

                    @pl.when(g < NGRP - 1)
                    def _():
                        fire_gather(sbuf[nxt].at[j - 4], j - 4)

            @pl.when(g < NGRP - 2)
            def _():
                fire_sidx(g + 2, par)

        def pair(gg, _):
            run_group(2 * gg, 0, True)
            run_group(2 * gg + 1, 1, False)
            return 0

        lax.fori_loop(0, NGRP // 2, pair, 0)
        for b in range(4, 8):
            drain_scatter(b)
        plsc.subcore_barrier()

        for off, size in _ROW_CHUNKS:
            row = r0 + off
            pltpu.sync_copy(acc.at[pl.ds(row, size)],
                            out_hbm.at[c, pl.ds(row, size)])

    return agg_kernel(h, src3d, dst3d)


_BLK = 632


def _row_spec(width):
    return pl.BlockSpec((_BLK, width), lambda i: (i, 0))


def _full_spec(r, ccols):
    return pl.BlockSpec((r, ccols), lambda i: (0, 0))


def _tc_prepare(x, ds0, ds1, dd0, dd1):
    def body(x_ref, a0, a1, b0, b1, h_ref, rss_ref, rsd_ref):
        rs = lax.rsqrt(jnp.maximum(a0[...] + a1[...], 1.0))
        rd = lax.rsqrt(jnp.maximum(b0[...] + b1[...], 1.0))
        rss_ref[...] = rs
        rsd_ref[...] = rd
        h_ref[...] = x_ref[...] * rs

    return pl.pallas_call(
        body,
        grid=(NPAD // _BLK,),
        in_specs=[
            _row_spec(D),
            _row_spec(1), _row_spec(1), _row_spec(1), _row_spec(1),
        ],
        out_specs=[_row_spec(D), _row_spec(1), _row_spec(1)],
        out_shape=[
            jax.ShapeDtypeStruct((NPAD, D), _f32),
            jax.ShapeDtypeStruct((NPAD, 1), _f32),
            jax.ShapeDtypeStruct((NPAD, 1), _f32),
        ],
    )(x, ds0, ds1, dd0, dd1)


def _tc_layer(p0, p1, rsd, rss, w, b):
    def body(a_ref, b_ref, rd_ref, rs_ref, w_ref, bias_ref, o_ref):
        z = (a_ref[...] + b_ref[...]) * rd_ref[...]
        z = jnp.dot(z, w_ref[...], preferred_element_type=_f32) + bias_ref[...]
        o_ref[...] = jnp.maximum(z, 0.0) * rs_ref[...]

    return pl.pallas_call(
        body,
        grid=(NPAD // _BLK,),
        in_specs=[
            _row_spec(D), _row_spec(D), _row_spec(1), _row_spec(1),
            _full_spec(D, D), _full_spec(1, D),
        ],
        out_specs=_row_spec(D),
        out_shape=jax.ShapeDtypeStruct((NPAD, D), _f32),
    )(p0, p1, rsd, rss, w, b)


def _tc_final(p0, p1, rsd, x, wres, bres, w3, b3, wop, bop):
    def body(a_ref, b_ref, rd_ref, x_ref, wr_ref, br_ref, w3_ref, b3_ref,
             wop_ref, bop_ref, o_ref):
        z = (a_ref[...] + b_ref[...]) * rd_ref[...]
        z = jnp.dot(z, w3_ref[...], preferred_element_type=_f32) + b3_ref[...]
        res = (jnp.dot(x_ref[...], wr_ref[...], preferred_element_type=_f32)
               + br_ref[...])
        h = jnp.maximum(z + res, 0.0)
        o_ref[...] = (
            jnp.dot(h, wop_ref[...], preferred_element_type=_f32)
            + bop_ref[...]
        )

    return pl.pallas_call(
        body,
        grid=(NPAD // _BLK,),
        in_specs=[
            _row_spec(D), _row_spec(D), _row_spec(1), _row_spec(D),
            _full_spec(D, D), _full_spec(1, D),
            _full_spec(D, D), _full_spec(1, D),
            _full_spec(D, NCLS), _full_spec(1, NCLS),
        ],
        out_specs=_row_spec(NCLS),
        out_shape=jax.ShapeDtypeStruct((NPAD, NCLS), _f32),
    )(p0, p1, rsd, x, wres, bres, w3, b3, wop, bop)


def kernel(inputs, edge_index, W1, b1, W2, b2, W3, b3, Wres, bres, Wop, bop):
    src = edge_index[0].astype(jnp.int32)
    dst = edge_index[1].astype(jnp.int32)

    pad_idx = N + (jnp.arange(EPAD - E, dtype=jnp.int32) % (NPAD - N))
    src3d = jnp.concatenate([src, pad_idx]).reshape(NW, RPW, CHUNK)
    dst3d = jnp.concatenate([dst, pad_idx]).reshape(NW, RPW, CHUNK)
    x = jnp.zeros((NPAD, D), _f32).at[:N].set(inputs)

    degs, degd = _sc_degrees(src3d, dst3d)
    ds0 = degs[:NPAD].reshape(NPAD, 1)
    ds1 = degs[NPAD:].reshape(NPAD, 1)
    dd0 = degd[:NPAD].reshape(NPAD, 1)
    dd1 = degd[NPAD:].reshape(NPAD, 1)

    h0, rss, rsd = _tc_prepare(x, ds0, ds1, dd0, dd1)

    p = _sc_aggregate(h0, src3d, dst3d)
    h1 = _tc_layer(p[0], p[1], rsd, rss, W1, b1.reshape(1, D))

    p = _sc_aggregate(h1, src3d, dst3d)
    h2 = _tc_layer(p[0], p[1], rsd, rss, W2, b2.reshape(1, D))

    p = _sc_aggregate(h2, src3d, dst3d)
    out = _tc_final(p[0], p[1], rsd, x, Wres, bres.reshape(1, D),
                    W3, b3.reshape(1, D), Wop, bop.reshape(1, NCLS))
    return out[:N]


# speedup vs baseline: 1.1683x; 1.1683x over previous
import functools

import jax
import jax.numpy as jnp
from jax import lax
from jax.experimental import pallas as pl
from jax.experimental.pallas import tpu as pltpu
from jax.experimental.pallas import tpu_sc as plsc

N = 10000
E = 320000
D = 128
NCLS = 40

NC = 2
NS = 16
NW = NC * NS
CHUNK = 80
RPW = 128
EPAD = NW * RPW * CHUNK
NPAD = 10112
RPT = NPAD // NS
NSLOT = 4
GRP = 8
NGRP = RPW // GRP
FILLB = 80


def _chunks(total, maxsz):
    out = [(i * maxsz, maxsz) for i in range(total // maxsz)]
    if total % maxsz:
        out.append(((total // maxsz) * maxsz, total % maxsz))
    return out


_ROW_CHUNKS = _chunks(RPT, CHUNK)

_f32 = jnp.float32


def _sc_degrees(src3d, dst3d):
    mesh = plsc.VectorSubcoreMesh(core_axis_name="c", subcore_axis_name="s")

    @functools.partial(
        pl.kernel,
        mesh=mesh,
        out_type=(
            jax.ShapeDtypeStruct((NC * NPAD,), _f32),
            jax.ShapeDtypeStruct((NC * NPAD,), _f32),
        ),
        scratch_types=[
            pltpu.VMEM((RPW, CHUNK), jnp.int32),
            pltpu.VMEM((RPW, CHUNK), jnp.int32),
            pltpu.VMEM((FILLB,), _f32),
            pltpu.VMEM((FILLB,), _f32),
            pltpu.VMEM_SHARED((NPAD,), _f32),
            pltpu.VMEM_SHARED((NPAD,), _f32),
            pltpu.SemaphoreType.DMA,
            pltpu.SemaphoreType.DMA,
        ],
    )
    def deg_kernel(src_hbm, dst_hbm, outs_hbm, outd_hbm,
                   sidx, didx, ones, zeros, acc_s, acc_d, sem_s, sem_d):
        c = lax.axis_index("c")
        s = lax.axis_index("s")
        w = c * NS + s

        pltpu.sync_copy(src_hbm.at[w], sidx)
        pltpu.sync_copy(dst_hbm.at[w], didx)
        one16 = jnp.full((16,), 1.0, _f32)
        zero16 = jnp.zeros((16,), _f32)

        def fill(i, _):
            ones[pl.ds(i * 16, 16)] = one16
            zeros[pl.ds(i * 16, 16)] = zero16
            return 0

        lax.fori_loop(0, FILLB // 16, fill, 0)

        r0 = s * RPT
        for off, size in _ROW_CHUNKS:
            pltpu.sync_copy(zeros.at[pl.ds(0, size)],
                            acc_s.at[pl.ds(r0 + off, size)])
            pltpu.sync_copy(zeros.at[pl.ds(0, size)],
                            acc_d.at[pl.ds(r0 + off, size)])
        plsc.subcore_barrier()

        GB = 8

        def group(g, _):
            for b in range(GB):
                r = g * GB + b
                pltpu.async_copy(ones.at[pl.ds(0, CHUNK)],
                                 acc_s.at[sidx.at[r]], sem_s, add=True)
                pltpu.async_copy(ones.at[pl.ds(0, CHUNK)],
                                 acc_d.at[didx.at[r]], sem_d, add=True)
            for b in range(GB):
                r = g * GB + b
                pltpu.make_async_copy(ones.at[pl.ds(0, CHUNK)],
                                      acc_s.at[sidx.at[r]], sem_s).wait()
                pltpu.make_async_copy(ones.at[pl.ds(0, CHUNK)],
                                      acc_d.at[didx.at[r]], sem_d).wait()
            return 0

        lax.fori_loop(0, RPW // GB, group, 0)
        plsc.subcore_barrier()

        for off, size in _ROW_CHUNKS:
            row = r0 + off
            pltpu.sync_copy(acc_s.at[pl.ds(row, size)], zeros.at[pl.ds(0, size)])
            pltpu.sync_copy(zeros.at[pl.ds(0, size)],
                            outs_hbm.at[pl.ds(c * NPAD + row, size)])
            pltpu.sync_copy(acc_d.at[pl.ds(row, size)], ones.at[pl.ds(0, size)])
            pltpu.sync_copy(ones.at[pl.ds(0, size)],
                            outd_hbm.at[pl.ds(c * NPAD + row, size)])

    return deg_kernel(src3d, dst3d)


def _sc_aggregate(h, src3d, dst3d):
    mesh = plsc.VectorSubcoreMesh(core_axis_name="c", subcore_axis_name="s")

    @functools.partial(
        pl.kernel,
        mesh=mesh,
        out_type=jax.ShapeDtypeStruct((NC, NPAD, D), _f32),
        scratch_types=(
            [pltpu.VMEM((GRP, CHUNK), jnp.int32) for _ in range(4)]
            + [pltpu.VMEM((CHUNK, D), _f32) for _ in range(NSLOT)]
            + [pltpu.VMEM_SHARED((NPAD, D), _f32)]
            + [pltpu.SemaphoreType.DMA for _ in range(NSLOT + 2)]
        ),
    )
    def agg_kernel(h_hbm, src_hbm, dst_hbm, out_hbm, *scr):
        sbuf = list(scr[0:2])
        dbuf = list(scr[2:4])
        rows = list(scr[4:4 + NSLOT])
        acc = scr[4 + NSLOT]
        gsem = list(scr[5 + NSLOT:5 + 2 * NSLOT])
        isem = list(scr[5 + 2 * NSLOT:7 + 2 * NSLOT])
        c = lax.axis_index("c")
        s = lax.axis_index("s")
        w = c * NS + s

        def fire_idx(grp, par, sync=False):
            srcslice = src_hbm.at[w, pl.ds(grp * GRP, GRP)]
            dstslice = dst_hbm.at[w, pl.ds(grp * GRP, GRP)]
            if sync:
                pltpu.sync_copy(srcslice, sbuf[par])
                pltpu.sync_copy(dstslice, dbuf[par])
            else:
                pltpu.async_copy(srcslice, sbuf[par], isem[par])
                pltpu.async_copy(dstslice, dbuf[par], isem[par])

        def wait_idx(par):
            pltpu.make_async_copy(src_hbm.at[w, pl.ds(0, GRP)],
                                  sbuf[par], isem[par]).wait()
            pltpu.make_async_copy(dst_hbm.at[w, pl.ds(0, GRP)],
                                  dbuf[par], isem[par]).wait()

        def fire_gather(sidx_row, b):
            pltpu.async_copy(h_hbm.at[sidx_row], rows[b], gsem[b])

        def wait_gather(b):
            pltpu.make_async_copy(h_hbm.at[pl.ds(0, CHUNK)],
                                  rows[b], gsem[b]).wait()

        zero16 = jnp.zeros((16,), _f32)

        def zr(i, _):
            for j in range(D // 16):
                rows[0][i, pl.ds(j * 16, 16)] = zero16
            return 0

        lax.fori_loop(0, CHUNK, zr, 0)
        r0 = s * RPT
        for off, size in _ROW_CHUNKS:
            pltpu.sync_copy(rows[0].at[pl.ds(0, size)],
                            acc.at[pl.ds(r0 + off, size)])
        plsc.subcore_barrier()

        fire_idx(0, 0, sync=True)
        fire_idx(1, 1)
        for b in range(NSLOT):
            fire_gather(sbuf[0].at[b], b)

        def run_group(g, par):
            nxt = 1 - par
            for j in range(GRP):
                if j == NSLOT:
                    @pl.when(g < NGRP - 1)
                    def _():
                        wait_idx(nxt)
                b = j % NSLOT
                wait_gather(b)
                pltpu.sync_copy(rows[b], acc.at[dbuf[par].at[j]], add=True)
                if j < NSLOT:
                    fire_gather(sbuf[par].at[j + NSLOT], b)
                else:
                    @pl.when(g < NGRP - 1)
                    def _():
                        fire_gather(sbuf[nxt].at[j - NSLOT], b)

            @pl.when(g < NGRP - 2)
            def _():
                fire_idx(g + 2, par)

        def pair(gg, _):
            run_group(2 * gg, 0)
            run_group(2 * gg + 1, 1)
            return 0

        lax.fori_loop(0, NGRP // 2, pair, 0)
        plsc.subcore_barrier()

        for off, size in _ROW_CHUNKS:
            row = r0 + off
            pltpu.sync_copy(acc.at[pl.ds(row, size)],
                            out_hbm.at[c, pl.ds(row, size)])

    return agg_kernel(h, src3d, dst3d)


_BLK = 632


def _row_spec(width):
    return pl.BlockSpec((_BLK, width), lambda i: (i, 0))


def _full_spec(r, ccols):
    return pl.BlockSpec((r, ccols), lambda i: (0, 0))


def _tc_prepare(x, ds0, ds1, dd0, dd1):
    def body(x_ref, a0, a1, b0, b1, h_ref, rss_ref, rsd_ref):
        rs = lax.rsqrt(jnp.maximum(a0[...] + a1[...], 1.0))
        rd = lax.rsqrt(jnp.maximum(b0[...] + b1[...], 1.0))
        rss_ref[...] = rs
        rsd_ref[...] = rd
        h_ref[...] = x_ref[...] * rs

    return pl.pallas_call(
        body,
        grid=(NPAD // _BLK,),
        in_specs=[
            _row_spec(D),
            _row_spec(1), _row_spec(1), _row_spec(1), _row_spec(1),
        ],
        out_specs=[_row_spec(D), _row_spec(1), _row_spec(1)],
        out_shape=[
            jax.ShapeDtypeStruct((NPAD, D), _f32),
            jax.ShapeDtypeStruct((NPAD, 1), _f32),
            jax.ShapeDtypeStruct((NPAD, 1), _f32),
        ],
    )(x, ds0, ds1, dd0, dd1)


def _tc_layer(p0, p1, rsd, rss, w, b):
    def body(a_ref, b_ref, rd_ref, rs_ref, w_ref, bias_ref, o_ref):
        z = (a_ref[...] + b_ref[...]) * rd_ref[...]
        z = jnp.dot(z, w_ref[...], preferred_element_type=_f32) + bias_ref[...]
        o_ref[...] = jnp.maximum(z, 0.0) * rs_ref[...]

    return pl.pallas_call(
        body,
        grid=(NPAD // _BLK,),
        in_specs=[
            _row_spec(D), _row_spec(D), _row_spec(1), _row_spec(1),
            _full_spec(D, D), _full_spec(1, D),
        ],
        out_specs=_row_spec(D),
        out_shape=jax.ShapeDtypeStruct((NPAD, D), _f32),
    )(p0, p1, rsd, rss, w, b)


def _tc_final(p0, p1, rsd, x, wres, bres, w3, b3, wop, bop):
    def body(a_ref, b_ref, rd_ref, x_ref, wr_ref, br_ref, w3_ref, b3_ref,
             wop_ref, bop_ref, o_ref):
        z = (a_ref[...] + b_ref[...]) * rd_ref[...]
        z = jnp.dot(z, w3_ref[...], preferred_element_type=_f32) + b3_ref[...]
        res = (jnp.dot(x_ref[...], wr_ref[...], preferred_element_type=_f32)
               + br_ref[...])
        h = jnp.maximum(z + res, 0.0)
        o_ref[...] = (
            jnp.dot(h, wop_ref[...], preferred_element_type=_f32)
            + bop_ref[...]
        )

    return pl.pallas_call(
        body,
        grid=(NPAD // _BLK,),
        in_specs=[
            _row_spec(D), _row_spec(D), _row_spec(1), _row_spec(D),
            _full_spec(D, D), _full_spec(1, D),
            _full_spec(D, D), _full_spec(1, D),
            _full_spec(D, NCLS), _full_spec(1, NCLS),
        ],
        out_specs=_row_spec(NCLS),
        out_shape=jax.ShapeDtypeStruct((NPAD, NCLS), _f32),
    )(p0, p1, rsd, x, wres, bres, w3, b3, wop, bop)


def kernel(inputs, edge_index, W1, b1, W2, b2, W3, b3, Wres, bres, Wop, bop):
    src = edge_index[0].astype(jnp.int32)
    dst = edge_index[1].astype(jnp.int32)

    pad_idx = N + (jnp.arange(EPAD - E, dtype=jnp.int32) % (NPAD - N))
    src3d = jnp.concatenate([src, pad_idx]).reshape(NW, RPW, CHUNK)
    dst3d = jnp.concatenate([dst, pad_idx]).reshape(NW, RPW, CHUNK)
    x = jnp.zeros((NPAD, D), _f32).at[:N].set(inputs)

    degs, degd = _sc_degrees(src3d, dst3d)
    ds0 = degs[:NPAD].reshape(NPAD, 1)
    ds1 = degs[NPAD:].reshape(NPAD, 1)
    dd0 = degd[:NPAD].reshape(NPAD, 1)
    dd1 = degd[NPAD:].reshape(NPAD, 1)

    h0, rss, rsd = _tc_prepare(x, ds0, ds1, dd0, dd1)

    p = _sc_aggregate(h0, src3d, dst3d)
    h1 = _tc_layer(p[0], p[1], rsd, rss, W1, b1.reshape(1, D))

    p = _sc_aggregate(h1, src3d, dst3d)
    h2 = _tc_layer(p[0], p[1], rsd, rss, W2, b2.reshape(1, D))

    p = _sc_aggregate(h2, src3d, dst3d)
    out = _tc_final(p[0], p[1], rsd, x, Wres, bres.reshape(1, D),
                    W3, b3.reshape(1, D), Wop, bop.reshape(1, NCLS))
    return out[:N]


# speedup vs baseline: 1.2301x; 1.0529x over previous
import functools

import jax
import jax.numpy as jnp
from jax import lax
from jax.experimental import pallas as pl
from jax.experimental.pallas import tpu as pltpu
from jax.experimental.pallas import tpu_sc as plsc

N = 10000
E = 320000
D = 128
NCLS = 40

NC = 2
NS = 16
NW = NC * NS
CHUNK = 80
RPW = 128
EPAD = NW * RPW * CHUNK
NPAD = 10112
RPT = NPAD // NS
NSLOT = 4
GRP = 8
NGRP = RPW // GRP
FILLB = 80


def _chunks(total, maxsz):
    out = [(i * maxsz, maxsz) for i in range(total // maxsz)]
    if total % maxsz:
        out.append(((total // maxsz) * maxsz, total % maxsz))
    return out


_ROW_CHUNKS = _chunks(RPT, CHUNK)

_f32 = jnp.float32


def _sc_degrees(src3d, dst3d):
    mesh = plsc.VectorSubcoreMesh(core_axis_name="c", subcore_axis_name="s")

    @functools.partial(
        pl.kernel,
        mesh=mesh,
        out_type=(
            jax.ShapeDtypeStruct((NC * NPAD,), _f32),
            jax.ShapeDtypeStruct((NC * NPAD,), _f32),
        ),
        scratch_types=[
            pltpu.VMEM((RPW, CHUNK), jnp.int32),
            pltpu.VMEM((RPW, CHUNK), jnp.int32),
            pltpu.VMEM((FILLB,), _f32),
            pltpu.VMEM((FILLB,), _f32),
            pltpu.VMEM_SHARED((NPAD,), _f32),
            pltpu.VMEM_SHARED((NPAD,), _f32),
            pltpu.SemaphoreType.DMA,
            pltpu.SemaphoreType.DMA,
        ],
    )
    def deg_kernel(src_hbm, dst_hbm, outs_hbm, outd_hbm,
                   sidx, didx, ones, zeros, acc_s, acc_d, sem_s, sem_d):
        c = lax.axis_index("c")
        s = lax.axis_index("s")
        w = c * NS + s

        pltpu.sync_copy(src_hbm.at[w], sidx)
        pltpu.sync_copy(dst_hbm.at[w], didx)
        one16 = jnp.full((16,), 1.0, _f32)
        zero16 = jnp.zeros((16,), _f32)

        def fill(i, _):
            ones[pl.ds(i * 16, 16)] = one16
            zeros[pl.ds(i * 16, 16)] = zero16
            return 0

        lax.fori_loop(0, FILLB // 16, fill, 0)

        r0 = s * RPT
        for off, size in _ROW_CHUNKS:
            pltpu.sync_copy(zeros.at[pl.ds(0, size)],
                            acc_s.at[pl.ds(r0 + off, size)])
            pltpu.sync_copy(zeros.at[pl.ds(0, size)],
                            acc_d.at[pl.ds(r0 + off, size)])
        plsc.subcore_barrier()

        GB = 8

        def group(g, _):
            for b in range(GB):
                r = g * GB + b
                pltpu.async_copy(ones.at[pl.ds(0, CHUNK)],
                                 acc_s.at[sidx.at[r]], sem_s, add=True)
                pltpu.async_copy(ones.at[pl.ds(0, CHUNK)],
                                 acc_d.at[didx.at[r]], sem_d, add=True)
            for b in range(GB):
                r = g * GB + b
                pltpu.make_async_copy(ones.at[pl.ds(0, CHUNK)],
                                      acc_s.at[sidx.at[r]], sem_s).wait()
                pltpu.make_async_copy(ones.at[pl.ds(0, CHUNK)],
                                      acc_d.at[didx.at[r]], sem_d).wait()
            return 0

        lax.fori_loop(0, RPW // GB, group, 0)
        plsc.subcore_barrier()

        for off, size in _ROW_CHUNKS:
            row = r0 + off
            pltpu.sync_copy(acc_s.at[pl.ds(row, size)], zeros.at[pl.ds(0, size)])
            pltpu.sync_copy(zeros.at[pl.ds(0, size)],
                            outs_hbm.at[pl.ds(c * NPAD + row, size)])
            pltpu.sync_copy(acc_d.at[pl.ds(row, size)], ones.at[pl.ds(0, size)])
            pltpu.sync_copy(ones.at[pl.ds(0, size)],
                            outd_hbm.at[pl.ds(c * NPAD + row, size)])

    return deg_kernel(src3d, dst3d)


def _sc_aggregate(h, src3d, dst3d):
    mesh = plsc.VectorSubcoreMesh(core_axis_name="c", subcore_axis_name="s")

    @functools.partial(
        pl.kernel,
        mesh=mesh,
        out_type=jax.ShapeDtypeStruct((NC, NPAD, D), _f32),
        scratch_types=(
            [pltpu.VMEM((GRP, CHUNK), jnp.int32) for _ in range(4)]
            + [pltpu.VMEM((CHUNK, D), _f32) for _ in range(NSLOT)]
            + [pltpu.VMEM_SHARED((NPAD, D), _f32)]
            + [pltpu.SemaphoreType.DMA for _ in range(NSLOT + 2)]
        ),
    )
    def agg_kernel(h_hbm, src_hbm, dst_hbm, out_hbm, *scr):
        sbuf = list(scr[0:2])
        dbuf = list(scr[2:4])
        rows = list(scr[4:4 + NSLOT])
        acc = scr[4 + NSLOT]
        gsem = list(scr[5 + NSLOT:5 + 2 * NSLOT])
        isem = list(scr[5 + 2 * NSLOT:7 + 2 * NSLOT])
        c = lax.axis_index("c")
        s = lax.axis_index("s")
        w = c * NS + s

        def fire_idx(grp, par, sync=False):
            srcslice = src_hbm.at[w, pl.ds(grp * GRP, GRP)]
            dstslice = dst_hbm.at[w, pl.ds(grp * GRP, GRP)]
            if sync:
                pltpu.sync_copy(srcslice, sbuf[par])
                pltpu.sync_copy(dstslice, dbuf[par])
            else:
                pltpu.async_copy(srcslice, sbuf[par], isem[par])
                pltpu.async_copy(dstslice, dbuf[par], isem[par])

        def wait_idx(par):
            pltpu.make_async_copy(src_hbm.at[w, pl.ds(0, GRP)],
                                  sbuf[par], isem[par]).wait()
            pltpu.make_async_copy(dst_hbm.at[w, pl.ds(0, GRP)],
                                  dbuf[par], isem[par]).wait()

        def fire_gather(sidx_row, b):
            pltpu.async_copy(h_hbm.at[sidx_row], rows[b], gsem[b])

        def wait_gather(b):
            pltpu.make_async_copy(h_hbm.at[pl.ds(0, CHUNK)],
                                  rows[b], gsem[b]).wait()

        zero16 = jnp.zeros((16,), _f32)

        def zr(i, _):
            for j in range(D // 16):
                rows[0][i, pl.ds(j * 16, 16)] = zero16
            return 0

        lax.fori_loop(0, CHUNK, zr, 0)
        r0 = s * RPT
        for off, size in _ROW_CHUNKS:
            pltpu.sync_copy(rows[0].at[pl.ds(0, size)],
                            acc.at[pl.ds(r0 + off, size)])
        plsc.subcore_barrier()

        fire_idx(0, 0, sync=True)
        fire_idx(1, 1)
        for b in range(NSLOT):
            fire_gather(sbuf[0].at[b], b)

        def run_group(g, par):
            nxt = 1 - par
            for j in range(GRP):
                if j == NSLOT:
                    @pl.when(g < NGRP - 1)
                    def _():
                        wait_idx(nxt)
                b = j % NSLOT
                wait_gather(b)
                pltpu.sync_copy(rows[b], acc.at[dbuf[par].at[j]], add=True)
                if j < NSLOT:
                    fire_gather(sbuf[par].at[j + NSLOT], b)
                else:
                    @pl.when(g < NGRP - 1)
                    def _():
                        fire_gather(sbuf[nxt].at[j - NSLOT], b)

            @pl.when(g < NGRP - 2)
            def _():
                fire_idx(g + 2, par)

        def pair(gg, _):
            run_group(2 * gg, 0)
            run_group(2 * gg + 1, 1)
            return 0

        lax.fori_loop(0, NGRP // 2, pair, 0)
        plsc.subcore_barrier()

        for off, size in _ROW_CHUNKS:
            row = r0 + off
            pltpu.sync_copy(acc.at[pl.ds(row, size)],
                            out_hbm.at[c, pl.ds(row, size)])

    return agg_kernel(h, src3d, dst3d)


_BLK = 632


def _row_spec(width):
    return pl.BlockSpec((_BLK, width), lambda i: (i, 0))


def _full_spec(r, ccols):
    return pl.BlockSpec((r, ccols), lambda i: (0, 0))


def _tc_prepare(x, degs, degd):
    def body(x_ref, a0, a1, b0, b1, h_ref, rss_ref, rsd_ref):
        rs = lax.rsqrt(jnp.maximum(a0[...] + a1[...], 1.0))
        rd = lax.rsqrt(jnp.maximum(b0[...] + b1[...], 1.0))
        rss_ref[...] = rs
        rsd_ref[...] = rd
        h_ref[...] = x_ref[...] * rs

    return pl.pallas_call(
        body,
        grid=(NPAD // _BLK,),
        in_specs=[
            _row_spec(D),
            _row_spec(1), _row_spec(1), _row_spec(1), _row_spec(1),
        ],
        out_specs=[_row_spec(D), _row_spec(1), _row_spec(1)],
        out_shape=[
            jax.ShapeDtypeStruct((NPAD, D), _f32),
            jax.ShapeDtypeStruct((NPAD, 1), _f32),
            jax.ShapeDtypeStruct((NPAD, 1), _f32),
        ],
    )(x, degs[:NPAD].reshape(NPAD, 1), degs[NPAD:].reshape(NPAD, 1),
      degd[:NPAD].reshape(NPAD, 1), degd[NPAD:].reshape(NPAD, 1))


def _part_spec(core):
    return pl.BlockSpec((1, _BLK, D), lambda i, c=core: (c, i, 0))


def _tc_layer(p, rsd, rss, w, b):
    def body(a_ref, b_ref, rd_ref, rs_ref, w_ref, bias_ref, o_ref):
        z = (a_ref[0] + b_ref[0]) * rd_ref[...]
        z = jnp.dot(z, w_ref[...], preferred_element_type=_f32) + bias_ref[...]
        o_ref[...] = jnp.maximum(z, 0.0) * rs_ref[...]

    return pl.pallas_call(
        body,
        grid=(NPAD // _BLK,),
        in_specs=[
            _part_spec(0), _part_spec(1), _row_spec(1), _row_spec(1),
            _full_spec(D, D), _full_spec(1, D),
        ],
        out_specs=_row_spec(D),
        out_shape=jax.ShapeDtypeStruct((NPAD, D), _f32),
    )(p, p, rsd, rss, w, b)


def _tc_final(p, rsd, x, wres, bres, w3, b3, wop, bop):
    def body(a_ref, b_ref, rd_ref, x_ref, wr_ref, br_ref, w3_ref, b3_ref,
             wop_ref, bop_ref, o_ref):
        z = (a_ref[0] + b_ref[0]) * rd_ref[...]
        z = jnp.dot(z, w3_ref[...], preferred_element_type=_f32) + b3_ref[...]
        res = (jnp.dot(x_ref[...], wr_ref[...], preferred_element_type=_f32)
               + br_ref[...])
        h = jnp.maximum(z + res, 0.0)
        o_ref[...] = (
            jnp.dot(h, wop_ref[...], preferred_element_type=_f32)
            + bop_ref[...]
        )

    return pl.pallas_call(
        body,
        grid=(NPAD // _BLK,),
        in_specs=[
            _part_spec(0), _part_spec(1), _row_spec(1), _row_spec(D),
            _full_spec(D, D), _full_spec(1, D),
            _full_spec(D, D), _full_spec(1, D),
            _full_spec(D, NCLS), _full_spec(1, NCLS),
        ],
        out_specs=_row_spec(NCLS),
        out_shape=jax.ShapeDtypeStruct((NPAD, NCLS), _f32),
    )(p, p, rsd, x, wres, bres, w3, b3, wop, bop)


def kernel(inputs, edge_index, W1, b1, W2, b2, W3, b3, Wres, bres, Wop, bop):
    src = edge_index[0].astype(jnp.int32)
    dst = edge_index[1].astype(jnp.int32)

    pad_idx = N + (jnp.arange(EPAD - E, dtype=jnp.int32) % (NPAD - N))
    src3d = jnp.concatenate([src, pad_idx]).reshape(NW, RPW, CHUNK)
    dst3d = jnp.concatenate([dst, pad_idx]).reshape(NW, RPW, CHUNK)
    x = jnp.zeros((NPAD, D), _f32).at[:N].set(inputs)

    degs, degd = _sc_degrees(src3d, dst3d)
    h0, rss, rsd = _tc_prepare(x, degs, degd)

    p = _sc_aggregate(h0, src3d, dst3d)
    h1 = _tc_layer(p, rsd, rss, W1, b1.reshape(1, D))

    p = _sc_aggregate(h1, src3d, dst3d)
    h2 = _tc_layer(p, rsd, rss, W2, b2.reshape(1, D))

    p = _sc_aggregate(h2, src3d, dst3d)
    out = _tc_final(p, rsd, x, Wres, bres.reshape(1, D),
                    W3, b3.reshape(1, D), Wop, bop.reshape(1, NCLS))
    return out[:N]


# speedup vs baseline: 1.2696x; 1.0321x over previous
import functools

import jax
import jax.numpy as jnp
from jax import lax
from jax.experimental import pallas as pl
from jax.experimental.pallas import tpu as pltpu
from jax.experimental.pallas import tpu_sc as plsc

N = 10000
E = 320000
D = 128
NCLS = 40

NC = 2
NS = 16
NW = NC * NS
CHUNK = 80
RPW = 128
EPAD = NW * RPW * CHUNK
NPAD = 10112
RPT = NPAD // NS
NSLOT = 4
GRP = 8
NGRP = RPW // GRP
FILLB = 80


def _chunks(total, maxsz):
    out = [(i * maxsz, maxsz) for i in range(total // maxsz)]
    if total % maxsz:
        out.append(((total // maxsz) * maxsz, total % maxsz))
    return out


_ROW_CHUNKS = _chunks(RPT, CHUNK)

_f32 = jnp.float32


def _sc_degrees(src3d, dst3d):
    mesh = plsc.VectorSubcoreMesh(core_axis_name="c", subcore_axis_name="s")

    @functools.partial(
        pl.kernel,
        mesh=mesh,
        out_type=(
            jax.ShapeDtypeStruct((NC * NPAD,), _f32),
            jax.ShapeDtypeStruct((NC * NPAD,), _f32),
        ),
        scratch_types=[
            pltpu.VMEM((RPW, CHUNK), jnp.int32),
            pltpu.VMEM((RPW, CHUNK), jnp.int32),
            pltpu.VMEM((FILLB,), _f32),
            pltpu.VMEM((FILLB,), _f32),
            pltpu.VMEM_SHARED((NPAD,), _f32),
            pltpu.VMEM_SHARED((NPAD,), _f32),
            pltpu.SemaphoreType.DMA,
            pltpu.SemaphoreType.DMA,
        ],
    )
    def deg_kernel(src_hbm, dst_hbm, outs_hbm, outd_hbm,
                   sidx, didx, ones, zeros, acc_s, acc_d, sem_s, sem_d):
        c = lax.axis_index("c")
        s = lax.axis_index("s")
        w = c * NS + s

        pltpu.sync_copy(src_hbm.at[w], sidx)
        pltpu.sync_copy(dst_hbm.at[w], didx)
        one16 = jnp.full((16,), 1.0, _f32)
        zero16 = jnp.zeros((16,), _f32)

        def fill(i, _):
            ones[pl.ds(i * 16, 16)] = one16
            zeros[pl.ds(i * 16, 16)] = zero16
            return 0

        lax.fori_loop(0, FILLB // 16, fill, 0)

        r0 = s * RPT
        for off, size in _ROW_CHUNKS:
            pltpu.sync_copy(zeros.at[pl.ds(0, size)],
                            acc_s.at[pl.ds(r0 + off, size)])
            pltpu.sync_copy(zeros.at[pl.ds(0, size)],
                            acc_d.at[pl.ds(r0 + off, size)])
        plsc.subcore_barrier()

        GB = 8

        def group(g, _):
            for b in range(GB):
                r = g * GB + b
                pltpu.async_copy(ones.at[pl.ds(0, CHUNK)],
                                 acc_s.at[sidx.at[r]], sem_s, add=True)
                pltpu.async_copy(ones.at[pl.ds(0, CHUNK)],
                                 acc_d.at[didx.at[r]], sem_d, add=True)
            for b in range(GB):
                r = g * GB + b
                pltpu.make_async_copy(ones.at[pl.ds(0, CHUNK)],
                                      acc_s.at[sidx.at[r]], sem_s).wait()
                pltpu.make_async_copy(ones.at[pl.ds(0, CHUNK)],
                                      acc_d.at[didx.at[r]], sem_d).wait()
            return 0

        lax.fori_loop(0, RPW // GB, group, 0)
        plsc.subcore_barrier()

        for off, size in _ROW_CHUNKS:
            row = r0 + off
            pltpu.sync_copy(acc_s.at[pl.ds(row, size)], zeros.at[pl.ds(0, size)])
            pltpu.sync_copy(zeros.at[pl.ds(0, size)],
                            outs_hbm.at[pl.ds(c * NPAD + row, size)])
            pltpu.sync_copy(acc_d.at[pl.ds(row, size)], ones.at[pl.ds(0, size)])
            pltpu.sync_copy(ones.at[pl.ds(0, size)],
                            outd_hbm.at[pl.ds(c * NPAD + row, size)])

    return deg_kernel(src3d, dst3d)


def _sc_aggregate(h, src3d, dst3d):
    mesh = plsc.VectorSubcoreMesh(core_axis_name="c", subcore_axis_name="s")

    @functools.partial(
        pl.kernel,
        mesh=mesh,
        out_type=jax.ShapeDtypeStruct((NC, NPAD, D), _f32),
        scratch_types=(
            [pltpu.VMEM((GRP, CHUNK), jnp.int32) for _ in range(4)]
            + [pltpu.VMEM((CHUNK, D), _f32) for _ in range(NSLOT)]
            + [pltpu.VMEM_SHARED((NPAD, D), _f32)]
            + [pltpu.SemaphoreType.DMA for _ in range(NSLOT + 2)]
        ),
    )
    def agg_kernel(h_hbm, src_hbm, dst_hbm, out_hbm, *scr):
        sbuf = list(scr[0:2])
        dbuf = list(scr[2:4])
        rows = list(scr[4:4 + NSLOT])
        acc = scr[4 + NSLOT]
        gsem = list(scr[5 + NSLOT:5 + 2 * NSLOT])
        isem = list(scr[5 + 2 * NSLOT:7 + 2 * NSLOT])
        c = lax.axis_index("c")
        s = lax.axis_index("s")
        w = c * NS + s

        def fire_idx(grp, par, sync=False):
            srcslice = src_hbm.at[w, pl.ds(grp * GRP, GRP)]
            dstslice = dst_hbm.at[w, pl.ds(grp * GRP, GRP)]
            if sync:
                pltpu.sync_copy(srcslice, sbuf[par])
                pltpu.sync_copy(dstslice, dbuf[par])
            else:
                pltpu.async_copy(srcslice, sbuf[par], isem[par])
                pltpu.async_copy(dstslice, dbuf[par], isem[par])

        def wait_idx(par):
            pltpu.make_async_copy(src_hbm.at[w, pl.ds(0, GRP)],
                                  sbuf[par], isem[par]).wait()
            pltpu.make_async_copy(dst_hbm.at[w, pl.ds(0, GRP)],
                                  dbuf[par], isem[par]).wait()

        def fire_gather(sidx_row, b):
            pltpu.async_copy(h_hbm.at[sidx_row], rows[b], gsem[b])

        def wait_gather(b):
            pltpu.make_async_copy(h_hbm.at[pl.ds(0, CHUNK)],
                                  rows[b], gsem[b]).wait()

        zero16 = jnp.zeros((16,), _f32)

        def zr(i, _):
            for j in range(D // 16):
                rows[0][i, pl.ds(j * 16, 16)] = zero16
            return 0

        lax.fori_loop(0, CHUNK, zr, 0)
        r0 = s * RPT
        for off, size in _ROW_CHUNKS:
            pltpu.sync_copy(rows[0].at[pl.ds(0, size)],
                            acc.at[pl.ds(r0 + off, size)])
        plsc.subcore_barrier()

        fire_idx(0, 0, sync=True)
        fire_idx(1, 1)
        for b in range(NSLOT):
            fire_gather(sbuf[0].at[b], b)

        def run_group(g, par):
            nxt = 1 - par
            for j in range(GRP):
                if j == NSLOT:
                    @pl.when(g < NGRP - 1)
                    def _():
                        wait_idx(nxt)
                b = j % NSLOT
                wait_gather(b)
                pltpu.sync_copy(rows[b], acc.at[dbuf[par].at[j]], add=True)
                if j < NSLOT:
                    fire_gather(sbuf[par].at[j + NSLOT], b)
                else:
                    @pl.when(g < NGRP - 1)
                    def _():
                        fire_gather(sbuf[nxt].at[j - NSLOT], b)

            @pl.when(g < NGRP - 2)
            def _():
                fire_idx(g + 2, par)

        def pair(gg, _):
            run_group(2 * gg, 0)
            run_group(2 * gg + 1, 1)
            return 0

        lax.fori_loop(0, NGRP // 2, pair, 0)
        plsc.subcore_barrier()

        for off, size in _ROW_CHUNKS:
            row = r0 + off
            pltpu.sync_copy(acc.at[pl.ds(row, size)],
                            out_hbm.at[c, pl.ds(row, size)])

    return agg_kernel(h, src3d, dst3d)


_BLK = 632


def _row_spec(width):
    return pl.BlockSpec((_BLK, width), lambda i: (i, 0))


def _full_spec(r, ccols):
    return pl.BlockSpec((r, ccols), lambda i: (0, 0))


def _tc_prepare(x, degs, degd):
    def body(x_ref, ds_ref, dd_ref, h_ref, rss_ref, rsd_ref):
        rs = lax.rsqrt(jnp.maximum(ds_ref[...], 1.0))
        rd = lax.rsqrt(jnp.maximum(dd_ref[...], 1.0))
        rss_ref[...] = rs
        rsd_ref[...] = rd
        h_ref[...] = x_ref[...] * rs

    dsb = jnp.broadcast_to((degs[:NPAD] + degs[NPAD:])[:, None], (NPAD, D))
    ddb = jnp.broadcast_to((degd[:NPAD] + degd[NPAD:])[:, None], (NPAD, D))
    return pl.pallas_call(
        body,
        grid=(NPAD // _BLK,),
        in_specs=[_row_spec(D), _row_spec(D), _row_spec(D)],
        out_specs=[_row_spec(D), _row_spec(D), _row_spec(D)],
        out_shape=[
            jax.ShapeDtypeStruct((NPAD, D), _f32),
            jax.ShapeDtypeStruct((NPAD, D), _f32),
            jax.ShapeDtypeStruct((NPAD, D), _f32),
        ],
    )(x, dsb, ddb)


def _part_spec(core):
    return pl.BlockSpec((1, _BLK, D), lambda i, c=core: (c, i, 0))


def _tc_layer(p, rsd, rss, w, b):
    def body(a_ref, b_ref, rd_ref, rs_ref, w_ref, bias_ref, o_ref):
        z = (a_ref[0] + b_ref[0]) * rd_ref[...]
        z = jnp.dot(z, w_ref[...], preferred_element_type=_f32) + bias_ref[...]
        o_ref[...] = jnp.maximum(z, 0.0) * rs_ref[...]

    return pl.pallas_call(
        body,
        grid=(NPAD // _BLK,),
        in_specs=[
            _part_spec(0), _part_spec(1), _row_spec(D), _row_spec(D),
            _full_spec(D, D), _full_spec(1, D),
        ],
        out_specs=_row_spec(D),
        out_shape=jax.ShapeDtypeStruct((NPAD, D), _f32),
    )(p, p, rsd, rss, w, b)


def _tc_final(p, rsd, x, wres, bres, w3, b3, wop, bop):
    def body(a_ref, b_ref, rd_ref, x_ref, wr_ref, br_ref, w3_ref, b3_ref,
             wop_ref, bop_ref, o_ref):
        z = (a_ref[0] + b_ref[0]) * rd_ref[...]
        z = jnp.dot(z, w3_ref[...], preferred_element_type=_f32) + b3_ref[...]
        res = (jnp.dot(x_ref[...], wr_ref[...], preferred_element_type=_f32)
               + br_ref[...])
        h = jnp.maximum(z + res, 0.0)
        o_ref[...] = (
            jnp.dot(h, wop_ref[...], preferred_element_type=_f32)
            + bop_ref[...]
        )

    return pl.pallas_call(
        body,
        grid=(NPAD // _BLK,),
        in_specs=[
            _part_spec(0), _part_spec(1), _row_spec(D), _row_spec(D),
            _full_spec(D, D), _full_spec(1, D),
            _full_spec(D, D), _full_spec(1, D),
            _full_spec(D, NCLS), _full_spec(1, NCLS),
        ],
        out_specs=_row_spec(NCLS),
        out_shape=jax.ShapeDtypeStruct((NPAD, NCLS), _f32),
    )(p, p, rsd, x, wres, bres, w3, b3, wop, bop)


def kernel(inputs, edge_index, W1, b1, W2, b2, W3, b3, Wres, bres, Wop, bop):
    src = edge_index[0].astype(jnp.int32)
    dst = edge_index[1].astype(jnp.int32)

    pad_idx = N + (jnp.arange(EPAD - E, dtype=jnp.int32) % (NPAD - N))
    src3d = jnp.concatenate([src, pad_idx]).reshape(NW, RPW, CHUNK)
    dst3d = jnp.concatenate([dst, pad_idx]).reshape(NW, RPW, CHUNK)
    x = jnp.zeros((NPAD, D), _f32).at[:N].set(inputs)

    degs, degd = _sc_degrees(src3d, dst3d)
    h0, rss, rsd = _tc_prepare(x, degs, degd)

    p = _sc_aggregate(h0, src3d, dst3d)
    h1 = _tc_layer(p, rsd, rss, W1, b1.reshape(1, D))

    p = _sc_aggregate(h1, src3d, dst3d)
    h2 = _tc_layer(p, rsd, rss, W2, b2.reshape(1, D))

    p = _sc_aggregate(h2, src3d, dst3d)
    out = _tc_final(p, rsd, x, Wres, bres.reshape(1, D),
                    W3, b3.reshape(1, D), Wop, bop.reshape(1, NCLS))
    return out[:N]


# speedup vs baseline: 1.2846x; 1.0118x over previous
import functools

import jax
import jax.numpy as jnp
from jax import lax
from jax.experimental import pallas as pl
from jax.experimental.pallas import tpu as pltpu
from jax.experimental.pallas import tpu_sc as plsc

N = 10000
E = 320000
D = 128
NCLS = 40

NC = 2
NS = 16
NW = NC * NS
CHUNK = 80
RPW = 128
EPAD = NW * RPW * CHUNK
NPAD = 10112
RPT = NPAD // NS
NSLOT = 4
GRP = 8
NGRP = RPW // GRP
FILLB = 80


def _chunks(total, maxsz):
    out = [(i * maxsz, maxsz) for i in range(total // maxsz)]
    if total % maxsz:
        out.append(((total // maxsz) * maxsz, total % maxsz))
    return out


_ROW_CHUNKS = _chunks(RPT, CHUNK)

_f32 = jnp.float32


def _sc_degrees(src3d, dst3d):
    mesh = plsc.VectorSubcoreMesh(core_axis_name="c", subcore_axis_name="s")

    @functools.partial(
        pl.kernel,
        mesh=mesh,
        out_type=(
            jax.ShapeDtypeStruct((NC * NPAD,), _f32),
            jax.ShapeDtypeStruct((NC * NPAD,), _f32),
        ),
        scratch_types=[
            pltpu.VMEM((RPW, CHUNK), jnp.int32),
            pltpu.VMEM((RPW, CHUNK), jnp.int32),
            pltpu.VMEM((FILLB,), _f32),
            pltpu.VMEM((FILLB,), _f32),
            pltpu.VMEM_SHARED((NPAD,), _f32),
            pltpu.VMEM_SHARED((NPAD,), _f32),
            pltpu.SemaphoreType.DMA,
            pltpu.SemaphoreType.DMA,
        ],
    )
    def deg_kernel(src_hbm, dst_hbm, outs_hbm, outd_hbm,
                   sidx, didx, ones, zeros, acc_s, acc_d, sem_s, sem_d):
        c = lax.axis_index("c")
        s = lax.axis_index("s")
        w = c * NS + s

        pltpu.sync_copy(src_hbm.at[w], sidx)
        pltpu.sync_copy(dst_hbm.at[w], didx)
        one16 = jnp.full((16,), 1.0, _f32)
        zero16 = jnp.zeros((16,), _f32)

        def fill(i, _):
            ones[pl.ds(i * 16, 16)] = one16
            zeros[pl.ds(i * 16, 16)] = zero16
            return 0

        lax.fori_loop(0, FILLB // 16, fill, 0)

        r0 = s * RPT
        for off, size in _ROW_CHUNKS:
            pltpu.sync_copy(zeros.at[pl.ds(0, size)],
                            acc_s.at[pl.ds(r0 + off, size)])
            pltpu.sync_copy(zeros.at[pl.ds(0, size)],
                            acc_d.at[pl.ds(r0 + off, size)])
        plsc.subcore_barrier()

        GB = 8

        def group(g, _):
            for b in range(GB):
                r = g * GB + b
                pltpu.async_copy(ones.at[pl.ds(0, CHUNK)],
                                 acc_s.at[sidx.at[r]], sem_s, add=True)
                pltpu.async_copy(ones.at[pl.ds(0, CHUNK)],
                                 acc_d.at[didx.at[r]], sem_d, add=True)
            for b in range(GB):
                r = g * GB + b
                pltpu.make_async_copy(ones.at[pl.ds(0, CHUNK)],
                                      acc_s.at[sidx.at[r]], sem_s).wait()
                pltpu.make_async_copy(ones.at[pl.ds(0, CHUNK)],
                                      acc_d.at[didx.at[r]], sem_d).wait()
            return 0

        lax.fori_loop(0, RPW // GB, group, 0)
        plsc.subcore_barrier()

        for off, size in _ROW_CHUNKS:
            row = r0 + off
            pltpu.sync_copy(acc_s.at[pl.ds(row, size)], zeros.at[pl.ds(0, size)])
            pltpu.sync_copy(zeros.at[pl.ds(0, size)],
                            outs_hbm.at[pl.ds(c * NPAD + row, size)])
            pltpu.sync_copy(acc_d.at[pl.ds(row, size)], ones.at[pl.ds(0, size)])
            pltpu.sync_copy(ones.at[pl.ds(0, size)],
                            outd_hbm.at[pl.ds(c * NPAD + row, size)])

    return deg_kernel(src3d, dst3d)


def _sc_aggregate(h, src3d, dst3d):
    mesh = plsc.VectorSubcoreMesh(core_axis_name="c", subcore_axis_name="s")

    @functools.partial(
        pl.kernel,
        mesh=mesh,
        out_type=jax.ShapeDtypeStruct((NC, NPAD, D), _f32),
        scratch_types=(
            [pltpu.VMEM((GRP, CHUNK), jnp.int32) for _ in range(4)]
            + [pltpu.VMEM((CHUNK, D), _f32) for _ in range(NSLOT)]
            + [pltpu.VMEM_SHARED((NPAD, D), _f32)]
            + [pltpu.SemaphoreType.DMA for _ in range(NSLOT + 2)]
        ),
    )
    def agg_kernel(h_hbm, src_hbm, dst_hbm, out_hbm, *scr):
        sbuf = list(scr[0:2])
        dbuf = list(scr[2:4])
        rows = list(scr[4:4 + NSLOT])
        acc = scr[4 + NSLOT]
        gsem = list(scr[5 + NSLOT:5 + 2 * NSLOT])
        isem = list(scr[5 + 2 * NSLOT:7 + 2 * NSLOT])
        c = lax.axis_index("c")
        s = lax.axis_index("s")
        w = c * NS + s

        def fire_idx(grp, par, sync=False):
            srcslice = src_hbm.at[w, pl.ds(grp * GRP, GRP)]
            dstslice = dst_hbm.at[w, pl.ds(grp * GRP, GRP)]
            if sync:
                pltpu.sync_copy(srcslice, sbuf[par])
                pltpu.sync_copy(dstslice, dbuf[par])
            else:
                pltpu.async_copy(srcslice, sbuf[par], isem[par])
                pltpu.async_copy(dstslice, dbuf[par], isem[par])

        def wait_idx(par):
            pltpu.make_async_copy(src_hbm.at[w, pl.ds(0, GRP)],
                                  sbuf[par], isem[par]).wait()
            pltpu.make_async_copy(dst_hbm.at[w, pl.ds(0, GRP)],
                                  dbuf[par], isem[par]).wait()

        def fire_gather(sidx_row, b):
            pltpu.async_copy(h_hbm.at[sidx_row], rows[b], gsem[b])

        def wait_gather(b):
            pltpu.make_async_copy(h_hbm.at[pl.ds(0, CHUNK)],
                                  rows[b], gsem[b]).wait()

        zero16 = jnp.zeros((16,), _f32)

        def zr(i, _):
            for j in range(D // 16):
                rows[0][i, pl.ds(j * 16, 16)] = zero16
            return 0

        lax.fori_loop(0, CHUNK, zr, 0)
        r0 = s * RPT
        for off, size in _ROW_CHUNKS:
            pltpu.sync_copy(rows[0].at[pl.ds(0, size)],
                            acc.at[pl.ds(r0 + off, size)])
        plsc.subcore_barrier()

        fire_idx(0, 0, sync=True)
        fire_idx(1, 1)
        for b in range(NSLOT):
            fire_gather(sbuf[0].at[b], b)

        def run_group(g, par):
            nxt = 1 - par
            for j in range(GRP):
                if j == NSLOT:
                    @pl.when(g < NGRP - 1)
                    def _():
                        wait_idx(nxt)
                b = j % NSLOT
                wait_gather(b)
                pltpu.sync_copy(rows[b], acc.at[dbuf[par].at[j]], add=True)
                if j < NSLOT:
                    fire_gather(sbuf[par].at[j + NSLOT], b)
                else:
                    @pl.when(g < NGRP - 1)
                    def _():
                        fire_gather(sbuf[nxt].at[j - NSLOT], b)

            @pl.when(g < NGRP - 2)
            def _():
                fire_idx(g + 2, par)

        def pair(gg, _):
            run_group(2 * gg, 0)
            run_group(2 * gg + 1, 1)
            return 0

        lax.fori_loop(0, NGRP // 2, pair, 0)
        plsc.subcore_barrier()

        for off, size in _ROW_CHUNKS:
            row = r0 + off
            pltpu.sync_copy(acc.at[pl.ds(row, size)],
                            out_hbm.at[c, pl.ds(row, size)])

    return agg_kernel(h, src3d, dst3d)


_BLK = 632


def _row_spec(width):
    return pl.BlockSpec((_BLK, width), lambda i: (i, 0))


def _full_spec(r, ccols):
    return pl.BlockSpec((r, ccols), lambda i: (0, 0))


def _tc_prepare(x, degs, degd):
    def body(x_ref, ds_ref, dd_ref, h_ref, rss_ref, rsd_ref):
        rs = lax.rsqrt(jnp.maximum(ds_ref[...], 1.0))
        rd = lax.rsqrt(jnp.maximum(dd_ref[...], 1.0))
        rss_ref[...] = rs
        rsd_ref[...] = rd
        h_ref[...] = x_ref[...] * rs

    dsb = jnp.broadcast_to((degs[:NPAD] + degs[NPAD:])[:, None], (NPAD, D))
    ddb = jnp.broadcast_to((degd[:NPAD] + degd[NPAD:])[:, None], (NPAD, D))
    return pl.pallas_call(
        body,
        grid=(NPAD // _BLK,),
        in_specs=[_row_spec(D), _row_spec(D), _row_spec(D)],
        out_specs=[_row_spec(D), _row_spec(D), _row_spec(D)],
        out_shape=[
            jax.ShapeDtypeStruct((NPAD, D), _f32),
            jax.ShapeDtypeStruct((NPAD, D), _f32),
            jax.ShapeDtypeStruct((NPAD, D), _f32),
        ],
    )(x, dsb, ddb)


def _part_spec(core):
    return pl.BlockSpec((1, _BLK, D), lambda i, c=core: (c, i, 0))


def _tc_layer(p, rsd, rss, w, b):
    def body(a_ref, b_ref, rd_ref, rs_ref, w_ref, bias_ref, o_ref):
        z = (a_ref[0] + b_ref[0]) * rd_ref[...]
        z = jnp.dot(z, w_ref[...], preferred_element_type=_f32) + bias_ref[...]
        o_ref[...] = jnp.maximum(z, 0.0) * rs_ref[...]

    return pl.pallas_call(
        body,
        grid=(NPAD // _BLK,),
        in_specs=[
            _part_spec(0), _part_spec(1), _row_spec(D), _row_spec(D),
            _full_spec(D, D), _full_spec(1, D),
        ],
        out_specs=_row_spec(D),
        out_shape=jax.ShapeDtypeStruct((NPAD, D), _f32),
    )(p, p, rsd, rss, w, b)


def _tc_final(p, rsd, x, wres, bres, w3, b3, wop, bop):
    def body(a_ref, b_ref, rd_ref, x_ref, wr_ref, br_ref, w3_ref, b3_ref,
             wop_ref, bop_ref, o_ref):
        z = (a_ref[0] + b_ref[0]) * rd_ref[...]
        z = jnp.dot(z, w3_ref[...], preferred_element_type=_f32) + b3_ref[...]
        res = (jnp.dot(x_ref[...], wr_ref[...], preferred_element_type=_f32)
               + br_ref[...])
        h = jnp.maximum(z + res, 0.0)
        o_ref[...] = (
            jnp.dot(h, wop_ref[...], preferred_element_type=_f32)
            + bop_ref[...]
        )

    fb = 1000
    prt = lambda c: pl.BlockSpec((1, fb, D), lambda i, c=c: (c, i, 0))
    row = lambda width: pl.BlockSpec((fb, width), lambda i: (i, 0))
    return pl.pallas_call(
        body,
        grid=(N // fb,),
        in_specs=[
            prt(0), prt(1), row(D), row(D),
            _full_spec(D, D), _full_spec(1, D),
            _full_spec(D, D), _full_spec(1, D),
            _full_spec(D, NCLS), _full_spec(1, NCLS),
        ],
        out_specs=row(NCLS),
        out_shape=jax.ShapeDtypeStruct((N, NCLS), _f32),
    )(p, p, rsd, x, wres, bres, w3, b3, wop, bop)


def kernel(inputs, edge_index, W1, b1, W2, b2, W3, b3, Wres, bres, Wop, bop):
    src = edge_index[0].astype(jnp.int32)
    dst = edge_index[1].astype(jnp.int32)

    pad_idx = N + (jnp.arange(EPAD - E, dtype=jnp.int32) % (NPAD - N))
    src3d = jnp.concatenate([src, pad_idx]).reshape(NW, RPW, CHUNK)
    dst3d = jnp.concatenate([dst, pad_idx]).reshape(NW, RPW, CHUNK)
    x = jnp.zeros((NPAD, D), _f32).at[:N].set(inputs)

    degs, degd = _sc_degrees(src3d, dst3d)
    h0, rss, rsd = _tc_prepare(x, degs, degd)

    p = _sc_aggregate(h0, src3d, dst3d)
    h1 = _tc_layer(p, rsd, rss, W1, b1.reshape(1, D))

    p = _sc_aggregate(h1, src3d, dst3d)
    h2 = _tc_layer(p, rsd, rss, W2, b2.reshape(1, D))

    p = _sc_aggregate(h2, src3d, dst3d)
    return _tc_final(p, rsd, x, Wres, bres.reshape(1, D),
                     W3, b3.reshape(1, D), Wop, bop.reshape(1, NCLS))
